# baseline (speedup 1.0000x reference)
import functools

import jax
import jax.numpy as jnp
from jax import lax
from jax.experimental import pallas as pl
from jax.experimental.pallas import tpu as pltpu
from jax.experimental.pallas import tpu_sc as plsc

SMOOTH = 0.1
L = 16
NBUF = 2
NROW = 1


def _token_stats(buf, r, label_vec, n_cols, unroll):
    n_it = n_cols // L // unroll

    m_init = tuple(jnp.full((L,), -jnp.inf, jnp.float32) for _ in range(unroll))
    s_init = tuple(jnp.zeros((L,), jnp.float32) for _ in range(unroll))

    @plsc.parallel_loop(0, n_it, carry=m_init + s_init)
    def pass_a(i, carry):
        ms = list(carry[:unroll])
        ss = list(carry[unroll:])
        for u in range(unroll):
            x = buf[r, pl.ds((i * unroll + u) * L, L)]
            ms[u] = jnp.maximum(ms[u], x)
            ss[u] = ss[u] + x
        return tuple(ms) + tuple(ss)

    mv = functools.reduce(jnp.maximum, pass_a[:unroll])
    sv = functools.reduce(jnp.add, pass_a[unroll:])

    @plsc.parallel_loop(0, n_it, carry=s_init)
    def pass_b(i, carry):
        es = list(carry)
        for u in range(unroll):
            x = buf[r, pl.ds((i * unroll + u) * L, L)]
            es[u] = es[u] + jnp.exp(x - mv)
        return tuple(es)

    ev = functools.reduce(jnp.add, pass_b)
    in_shard = label_vec >= 0
    idx = jnp.maximum(label_vec, 0)
    row_splat = jnp.full((L,), r, jnp.int32)
    xl_v = jnp.where(in_shard, plsc.load_gather(buf, [row_splat, idx]), 0.0)

    lane = jnp.arange(L, dtype=jnp.int32)

    def allreduce(v, op):
        for d in (1, 2, 4, 8):
            v = op(v, v.at[lane ^ d].get(mode="promise_in_bounds"))
        return v

    m_all = allreduce(mv, jnp.maximum)
    s_all = allreduce(sv, jnp.add)
    e_all = allreduce(ev * jnp.exp(mv - m_all), jnp.add)
    return m_all, s_all, e_all, xl_v


def _sc_body(n_tokens, col0, n_cols, tpw, unroll,
             preds_hbm, labels_hbm, part_hbm,
             rows, lab_v, res_v, sems):
    c = lax.axis_index("c")
    s = lax.axis_index("s")
    info = plsc.get_sparse_core_info()
    wid = s * info.num_cores + c
    base = wid * tpw
    lane0 = jnp.arange(L, dtype=jnp.int32) == 0

    pltpu.sync_copy(labels_hbm.at[pl.ds(base, tpw)], lab_v)

    def grp_src(g):
        t0 = jnp.minimum(base + g * NROW, n_tokens - NROW)
        return preds_hbm.at[pl.ds(t0, NROW), pl.ds(col0, n_cols)]

    for u in range(NBUF):
        pltpu.async_copy(grp_src(u), rows[u], sems[u])

    def handle_group(g, buf, sem):
        pltpu.make_async_copy(grp_src(g), buf, sem).wait()
        for r in range(NROW):
            k = g * NROW + r
            label_vec = plsc.load_gather(lab_v, [jnp.full((L,), k, jnp.int32)])
            stats = _token_stats(buf, r, label_vec - col0, n_cols, unroll)
            row = jnp.full((L,), k, jnp.int32)
            for q, vec in enumerate(stats):
                plsc.store_scatter(res_v, [row, jnp.full((L,), q, jnp.int32)],
                                   vec, mask=lane0)
        pltpu.async_copy(grp_src(g + NBUF), buf, sem)

    def body(j, carry):
        for u in range(NBUF):
            handle_group(NBUF * j + u, rows[u], sems[u])
        return carry

    lax.fori_loop(0, tpw // (NROW * NBUF), body, jnp.int32(0))
    for u in range(NBUF):
        pltpu.make_async_copy(grp_src(0), rows[u], sems[u]).wait()

    pltpu.sync_copy(res_v, part_hbm.at[pl.ds(base, tpw), :])


def _tc_block(preds_ref, labels_ref, part_ref):
    x = preds_ref[...]
    m = jnp.max(x, axis=1)
    e = jnp.sum(jnp.exp(x - m[:, None]), axis=1)
    total = jnp.sum(x, axis=1)
    labels = labels_ref[0, 0, :]
    col = jax.lax.broadcasted_iota(jnp.int32, x.shape, 1)
    xl = jnp.sum(jnp.where(col == labels[:, None], x, 0.0), axis=1)
    part_ref[...] = jnp.concatenate(
        [m[:, None], total[:, None], e[:, None], xl[:, None]], axis=1)


def _combine_block(tc_ref, sc_ref, out_ref, *, n_tokens, num_classes):
    tc = tc_ref[...]
    m1, s1, e1, x1 = tc[:, 0:1], tc[:, 1:2], tc[:, 2:3], tc[:, 3:4]
    sc = sc_ref[...]
    m2, s2 = sc[:, 0:1], sc[:, 1:2]
    e2, x2 = sc[:, 2:3], sc[:, 3:4]
    m = jnp.maximum(m1, m2)
    sumexp = e1 * jnp.exp(m1 - m) + e2 * jnp.exp(m2 - m)
    total = s1 + s2
    xl = x1 + x2
    lse = m + jnp.log(sumexp)
    a = SMOOTH / (num_classes - 1)
    lp_label = xl - lse
    sum_lp = total - num_classes * lse
    loss_t = -(a * (sum_lp - lp_label) + (1.0 - SMOOTH) * lp_label)
    out_ref[...] = jnp.sum(loss_t, keepdims=True).reshape(1, 1) / n_tokens


def kernel(preds, labels):
    b, t, c = preds.shape
    n_tokens = b * t
    preds2 = preds.reshape(n_tokens, c)
    labels1 = labels.reshape(n_tokens).astype(jnp.int32)

    sc_cols = 8192
    ck = c - sc_cols
    tb = 256

    info = plsc.get_sparse_core_info()
    n_workers = info.num_cores * info.num_subcores
    tpw = n_tokens // n_workers
    unroll = 8

    mesh = plsc.VectorSubcoreMesh(core_axis_name="c", subcore_axis_name="s")
    sc_call = pl.kernel(
        functools.partial(_sc_body, n_tokens, ck, sc_cols, tpw, unroll),
        out_type=jax.ShapeDtypeStruct((n_tokens, 4), jnp.float32),
        mesh=mesh,
        compiler_params=pltpu.CompilerParams(needs_layout_passes=False),
        scratch_types=[
            [pltpu.VMEM((NROW, sc_cols), jnp.float32) for _ in range(NBUF)],
            pltpu.VMEM((tpw,), jnp.int32),
            pltpu.VMEM((tpw, 4), jnp.float32),
            [pltpu.SemaphoreType.DMA for _ in range(NBUF)],
        ],
    )
    sc_part = sc_call(preds2, labels1)

    n_blocks = n_tokens // tb
    labels3 = labels1.reshape(n_blocks, 1, tb)
    tc_part = pl.pallas_call(
        _tc_block,
        grid=(n_blocks,),
        in_specs=[
            pl.BlockSpec((tb, ck), lambda i: (i, 0)),
            pl.BlockSpec((1, 1, tb), lambda i: (i, 0, 0)),
        ],
        out_specs=pl.BlockSpec((tb, 4), lambda i: (i, 0)),
        out_shape=jax.ShapeDtypeStruct((n_tokens, 4), jnp.float32),
    )(preds2, labels3)

    out = pl.pallas_call(
        functools.partial(_combine_block, n_tokens=n_tokens, num_classes=c),
        out_shape=jax.ShapeDtypeStruct((1, 1), jnp.float32),
    )(tc_part, sc_part)
    return out[0, 0]


# speedup vs baseline: 1.0063x; 1.0063x over previous
import functools

import jax
import jax.numpy as jnp
from jax import lax
from jax.experimental import pallas as pl
from jax.experimental.pallas import tpu as pltpu
from jax.experimental.pallas import tpu_sc as plsc

SMOOTH = 0.1
L = 16
NBUF = 2
NROW = 1


def _token_stats(buf, r, label_vec, n_cols, unroll):
    n_it = n_cols // L // unroll

    m_init = tuple(jnp.full((L,), -jnp.inf, jnp.float32) for _ in range(unroll))
    s_init = tuple(jnp.zeros((L,), jnp.float32) for _ in range(unroll))

    @plsc.parallel_loop(0, n_it, carry=m_init + s_init)
    def pass_a(i, carry):
        ms = list(carry[:unroll])
        ss = list(carry[unroll:])
        for u in range(unroll):
            x = buf[r, pl.ds((i * unroll + u) * L, L)]
            ms[u] = jnp.maximum(ms[u], x)
            ss[u] = ss[u] + x
        return tuple(ms) + tuple(ss)

    mv = functools.reduce(jnp.maximum, pass_a[:unroll])
    sv = functools.reduce(jnp.add, pass_a[unroll:])

    @plsc.parallel_loop(0, n_it, carry=s_init)
    def pass_b(i, carry):
        es = list(carry)
        for u in range(unroll):
            x = buf[r, pl.ds((i * unroll + u) * L, L)]
            es[u] = es[u] + jnp.exp(x - mv)
        return tuple(es)

    ev = functools.reduce(jnp.add, pass_b)
    in_shard = label_vec >= 0
    idx = jnp.maximum(label_vec, 0)
    row_splat = jnp.full((L,), r, jnp.int32)
    xl_v = jnp.where(in_shard, plsc.load_gather(buf, [row_splat, idx]), 0.0)

    lane = jnp.arange(L, dtype=jnp.int32)

    def allreduce(v, op):
        for d in (1, 2, 4, 8):
            v = op(v, v.at[lane ^ d].get(mode="promise_in_bounds"))
        return v

    m_all = allreduce(mv, jnp.maximum)
    s_all = allreduce(sv, jnp.add)
    e_all = allreduce(ev * jnp.exp(mv - m_all), jnp.add)
    return m_all, s_all, e_all, xl_v


def _sc_body(n_tokens, col0, n_cols, tpw, unroll,
             preds_hbm, labels_hbm, part_hbm,
             rows, lab_v, res_v, sems):
    c = lax.axis_index("c")
    s = lax.axis_index("s")
    info = plsc.get_sparse_core_info()
    wid = s * info.num_cores + c
    base = wid * tpw
    lane0 = jnp.arange(L, dtype=jnp.int32) == 0

    pltpu.sync_copy(labels_hbm.at[pl.ds(base, tpw)], lab_v)

    def grp_src(g):
        t0 = jnp.minimum(base + g * NROW, n_tokens - NROW)
        return preds_hbm.at[pl.ds(t0, NROW), pl.ds(col0, n_cols)]

    for u in range(NBUF):
        pltpu.async_copy(grp_src(u), rows[u], sems[u])

    def handle_group(g, buf, sem):
        pltpu.make_async_copy(grp_src(g), buf, sem).wait()
        for r in range(NROW):
            k = g * NROW + r
            label_vec = plsc.load_gather(lab_v, [jnp.full((L,), k, jnp.int32)])
            stats = _token_stats(buf, r, label_vec - col0, n_cols, unroll)
            row = jnp.full((L,), k, jnp.int32)
            for q, vec in enumerate(stats):
                plsc.store_scatter(res_v, [row, jnp.full((L,), q, jnp.int32)],
                                   vec, mask=lane0)
        pltpu.async_copy(grp_src(g + NBUF), buf, sem)

    def body(j, carry):
        for u in range(NBUF):
            handle_group(NBUF * j + u, rows[u], sems[u])
        return carry

    lax.fori_loop(0, tpw // (NROW * NBUF), body, jnp.int32(0))
    for u in range(NBUF):
        pltpu.make_async_copy(grp_src(0), rows[u], sems[u]).wait()

    pltpu.sync_copy(res_v, part_hbm.at[pl.ds(base, tpw), :])


def _tc_block(preds_ref, labels_ref, part_ref):
    x = preds_ref[...]
    m = jnp.max(x, axis=1)
    e = jnp.sum(jnp.exp(x - m[:, None]), axis=1)
    total = jnp.sum(x, axis=1)
    labels = labels_ref[0, 0, :]
    col = jax.lax.broadcasted_iota(jnp.int32, x.shape, 1)
    xl = jnp.sum(jnp.where(col == labels[:, None], x, 0.0), axis=1)
    part_ref[...] = jnp.concatenate(
        [m[:, None], total[:, None], e[:, None], xl[:, None]], axis=1)


def _combine_block(tc_ref, sc_ref, out_ref, *, n_tokens, num_classes):
    tc = tc_ref[...]
    m1, s1, e1, x1 = tc[:, 0:1], tc[:, 1:2], tc[:, 2:3], tc[:, 3:4]
    sc = sc_ref[...]
    m2, s2 = sc[:, 0:1], sc[:, 1:2]
    e2, x2 = sc[:, 2:3], sc[:, 3:4]
    m = jnp.maximum(m1, m2)
    sumexp = e1 * jnp.exp(m1 - m) + e2 * jnp.exp(m2 - m)
    total = s1 + s2
    xl = x1 + x2
    lse = m + jnp.log(sumexp)
    a = SMOOTH / (num_classes - 1)
    lp_label = xl - lse
    sum_lp = total - num_classes * lse
    loss_t = -(a * (sum_lp - lp_label) + (1.0 - SMOOTH) * lp_label)
    out_ref[...] = jnp.sum(loss_t, keepdims=True).reshape(1, 1) / n_tokens


def kernel(preds, labels):
    b, t, c = preds.shape
    n_tokens = b * t
    preds2 = preds.reshape(n_tokens, c)
    labels1 = labels.reshape(n_tokens).astype(jnp.int32)

    sc_cols = 8064
    ck = c - sc_cols
    tb = 256

    info = plsc.get_sparse_core_info()
    n_workers = info.num_cores * info.num_subcores
    tpw = n_tokens // n_workers
    unroll = 8

    mesh = plsc.VectorSubcoreMesh(core_axis_name="c", subcore_axis_name="s")
    sc_call = pl.kernel(
        functools.partial(_sc_body, n_tokens, ck, sc_cols, tpw, unroll),
        out_type=jax.ShapeDtypeStruct((n_tokens, 4), jnp.float32),
        mesh=mesh,
        compiler_params=pltpu.CompilerParams(needs_layout_passes=False),
        scratch_types=[
            [pltpu.VMEM((NROW, sc_cols), jnp.float32) for _ in range(NBUF)],
            pltpu.VMEM((tpw,), jnp.int32),
            pltpu.VMEM((tpw, 4), jnp.float32),
            [pltpu.SemaphoreType.DMA for _ in range(NBUF)],
        ],
    )
    sc_part = sc_call(preds2, labels1)

    n_blocks = n_tokens // tb
    labels3 = labels1.reshape(n_blocks, 1, tb)
    tc_part = pl.pallas_call(
        _tc_block,
        grid=(n_blocks,),
        in_specs=[
            pl.BlockSpec((tb, ck), lambda i: (i, 0)),
            pl.BlockSpec((1, 1, tb), lambda i: (i, 0, 0)),
        ],
        out_specs=pl.BlockSpec((tb, 4), lambda i: (i, 0)),
        out_shape=jax.ShapeDtypeStruct((n_tokens, 4), jnp.float32),
    )(preds2, labels3)

    out = pl.pallas_call(
        functools.partial(_combine_block, n_tokens=n_tokens, num_classes=c),
        out_shape=jax.ShapeDtypeStruct((1, 1), jnp.float32),
    )(tc_part, sc_part)
    return out[0, 0]
